# Initial kernel scaffold; baseline (speedup 1.0000x reference)
import jax
import jax.numpy as jnp
from jax.experimental import pallas as pl


def kernel(x, edge_attr, edge_index, ne_w1, ne_b1, ne_w2, ne_b2, ee_w1, ee_b1, ee_w2, ee_b2, gn_em_w1, gn_em_b1, gn_em_w2, gn_em_b2, gn_nm_w1, gn_nm_b1, gn_nm_w2, gn_nm_b2, de_w1, de_b1, de_w2, de_b2, node_mean, node_std, edge_mean, edge_std, out_mean, out_std):
    raise NotImplementedError("write your pallas kernel here")


# speedup vs baseline: 2.8789x; 2.8789x over previous
import functools

import jax
import jax.numpy as jnp
from jax import lax
from jax.experimental import pallas as pl
from jax.experimental.pallas import tpu as pltpu
from jax.experimental.pallas import tpu_sc as plsc

N = 10000
E = 320000
H = 128

NC = 2
NS = 16
NW = NC * NS
K = 80
G = 5
CH = K * G
EPW = E // NW
STEPS = EPW // CH
NH = N // NC
NACC = 5120
RPT = NACC // NS
TRASH = NH
ET = E // NS
STEPS_S = ET // CH

BN = 1000
BE = 2000

_f32 = jnp.float32


def _dot(a, b):
    return jnp.dot(a, b, preferred_element_type=_f32)


def _rows_spec(bs, w):
    return pl.BlockSpec((bs, w), lambda i: (i, 0))


def _full_spec(r, c):
    return pl.BlockSpec((r, c), lambda i: (0, 0))


def _node_encode_body(x_ref, mean_ref, std_ref, w1_ref, b1_ref, w2_ref, b2_ref,
                      ws_ref, bs_ref, wd_ref, x0_ref, ps_ref, pd_ref):
    xn = (x_ref[...] - mean_ref[...]) / std_ref[...]
    h = jnp.maximum(_dot(xn, w1_ref[...]) + b1_ref[...], 0.0)
    x0 = _dot(h, w2_ref[...]) + b2_ref[...]
    x0_ref[...] = x0
    ps_ref[...] = _dot(x0, ws_ref[...]) + bs_ref[...]
    pd_ref[...] = _dot(x0, wd_ref[...])


def _node_encode(x, mean, std, w1, b1, w2, b2, ws, bs, wd):
    return pl.pallas_call(
        _node_encode_body,
        grid=(N // BN,),
        in_specs=[_rows_spec(BN, H), _full_spec(1, H), _full_spec(1, H),
                  _full_spec(H, H), _full_spec(1, H), _full_spec(H, H),
                  _full_spec(1, H), _full_spec(H, H), _full_spec(1, H),
                  _full_spec(H, H)],
        out_specs=[_rows_spec(BN, H)] * 3,
        out_shape=[jax.ShapeDtypeStruct((N, H), _f32)] * 3,
    )(x, mean, std, w1, b1, w2, b2, ws, bs, wd)


def _edge_encode_body(e_ref, mean_ref, std_ref, w1_ref, b1_ref, w2_ref, b2_ref,
                      out_ref):
    en = (e_ref[...] - mean_ref[...]) / std_ref[...]
    h = jnp.maximum(_dot(en, w1_ref[...]) + b1_ref[...], 0.0)
    out_ref[...] = _dot(h, w2_ref[...]) + b2_ref[...]


def _edge_encode(e, mean, std, w1, b1, w2, b2):
    din = e.shape[1]
    return pl.pallas_call(
        _edge_encode_body,
        grid=(E // BE,),
        in_specs=[_rows_spec(BE, din), _full_spec(1, din), _full_spec(1, din),
                  _full_spec(din, H), _full_spec(1, H), _full_spec(H, H),
                  _full_spec(1, H)],
        out_specs=_rows_spec(BE, H),
        out_shape=jax.ShapeDtypeStruct((E, H), _f32),
    )(e, mean, std, w1, b1, w2, b2)


def _edge_update_body(e_ref, gs_ref, gd_ref, we_ref, w2_ref, b2_ref, out_ref):
    h = jnp.maximum(_dot(e_ref[...], we_ref[...]) + gs_ref[...] + gd_ref[...],
                    0.0)
    out_ref[...] = _dot(h, w2_ref[...]) + b2_ref[...]


def _edge_update(e, gs, gd, we, w2, b2):
    return pl.pallas_call(
        _edge_update_body,
        grid=(E // BE,),
        in_specs=[_rows_spec(BE, H), _rows_spec(BE, H), _rows_spec(BE, H),
                  _full_spec(H, H), _full_spec(H, H), _full_spec(1, H)],
        out_specs=_rows_spec(BE, H),
        out_shape=jax.ShapeDtypeStruct((E, H), _f32),
    )(e, gs, gd, we, w2, b2)


def _node_update_body(x_ref, agg_ref, a1_ref, a2_ref, b1_ref, w2_ref,
                      b2_ref, ws_ref, bs_ref, wd_ref, x1_ref, ps_ref, pd_ref):
    agg = agg_ref[...]
    h = jnp.maximum(_dot(x_ref[...], a1_ref[...]) + _dot(agg, a2_ref[...])
                    + b1_ref[...], 0.0)
    x1 = _dot(h, w2_ref[...]) + b2_ref[...]
    x1_ref[...] = x1
    ps_ref[...] = _dot(x1, ws_ref[...]) + bs_ref[...]
    pd_ref[...] = _dot(x1, wd_ref[...])


def _node_update(x, agg, a1, a2, b1, w2, b2, ws, bs, wd):
    return pl.pallas_call(
        _node_update_body,
        grid=(N // BN,),
        in_specs=[_rows_spec(BN, H)] * 2 + [
            _full_spec(H, H), _full_spec(H, H), _full_spec(1, H),
            _full_spec(H, H), _full_spec(1, H), _full_spec(H, H),
            _full_spec(1, H), _full_spec(H, H)],
        out_specs=[_rows_spec(BN, H)] * 3,
        out_shape=[jax.ShapeDtypeStruct((N, H), _f32)] * 3,
    )(x, agg, a1, a2, b1, w2, b2, ws, bs, wd)


def _node_final_body(x_ref, agg_ref, x0_ref, a1_ref, a2_ref, b1_ref,
                     w2_ref, b2_ref, dw1_ref, db1_ref, dw2_ref, db2_ref,
                     std_ref, mean_ref, out_ref):
    agg = agg_ref[...]
    h = jnp.maximum(_dot(x_ref[...], a1_ref[...]) + _dot(agg, a2_ref[...])
                    + b1_ref[...], 0.0)
    x2 = _dot(h, w2_ref[...]) + b2_ref[...]
    xf = jnp.tanh(x2 + x0_ref[...])
    hh = jnp.maximum(_dot(xf, dw1_ref[...]) + db1_ref[...], 0.0)
    o = _dot(hh, dw2_ref[...]) + db2_ref[...]
    out_ref[...] = o * std_ref[...] + mean_ref[...]


def _node_final(x, agg, x0, a1, a2, b1, w2, b2, dw1, db1, dw2p, db2p,
                stdp, meanp):
    return pl.pallas_call(
        _node_final_body,
        grid=(N // BN,),
        in_specs=[_rows_spec(BN, H)] * 3 + [
            _full_spec(H, H), _full_spec(H, H), _full_spec(1, H),
            _full_spec(H, H), _full_spec(1, H), _full_spec(H, H),
            _full_spec(1, H), _full_spec(H, H), _full_spec(1, H),
            _full_spec(1, H), _full_spec(1, H)],
        out_specs=_rows_spec(BN, H),
        out_shape=jax.ShapeDtypeStruct((N, H), _f32),
    )(x, agg, x0, a1, a2, b1, w2, b2, dw1, db1, dw2p, db2p, stdp, meanp)


@functools.cache
def _build_gather_pair():
    mesh = plsc.VectorSubcoreMesh(core_axis_name="c", subcore_axis_name="s")

    @functools.partial(
        pl.kernel,
        mesh=mesh,
        out_type=[jax.ShapeDtypeStruct((E, H), _f32),
                  jax.ShapeDtypeStruct((E, H), _f32)],
        scratch_types=[pltpu.VMEM((G, K), jnp.int32),
                       pltpu.VMEM((G, K), jnp.int32),
                       pltpu.VMEM((CH, H), _f32),
                       pltpu.VMEM((CH, H), _f32),
                       pltpu.SemaphoreType.DMA],
    )
    def gather_pair(ps_hbm, pd_hbm, src_hbm, dst_hbm, gs_hbm, gd_hbm,
                    si_v, di_v, rs_v, rd_v, sem):
        cid = lax.axis_index("c")
        sid = lax.axis_index("s")
        wid = cid * NS + sid

        def step(j, carry):
            e0 = wid * EPW + j * CH
            pltpu.sync_copy(src_hbm.at[wid, j], si_v)
            pltpu.sync_copy(dst_hbm.at[wid, j], di_v)
            cps = []
            for g in range(G):
                cps.append(pltpu.async_copy(ps_hbm.at[si_v.at[g]],
                                            rs_v.at[pl.ds(g * K, K)], sem))
                cps.append(pltpu.async_copy(pd_hbm.at[di_v.at[g]],
                                            rd_v.at[pl.ds(g * K, K)], sem))
            for c in cps:
                c.wait()
            pltpu.sync_copy(rs_v, gs_hbm.at[pl.ds(e0, CH)])
            pltpu.sync_copy(rd_v, gd_hbm.at[pl.ds(e0, CH)])
            return carry

        lax.fori_loop(0, STEPS, step, 0)

    return gather_pair


def _gather_pair(ps, pd, src2, dst2):
    return _build_gather_pair()(ps, pd, src2, dst2)


@functools.cache
def _build_segment_sum2():
    mesh = plsc.VectorSubcoreMesh(core_axis_name="c", subcore_axis_name="s")

    @functools.partial(
        pl.kernel,
        mesh=mesh,
        out_type=jax.ShapeDtypeStruct((2 * NACC, H), _f32),
        scratch_types=[pltpu.VMEM((G, K), jnp.int32),
                       pltpu.VMEM((G, K), jnp.int32),
                       pltpu.VMEM((CH, H), _f32),
                       pltpu.VMEM_SHARED((NACC, H), _f32),
                       pltpu.SemaphoreType.DMA],
    )
    def segment_sum2(vals_hbm, dst_hbm, zeros_hbm, out_hbm,
                     di_v, ti_v, rows_v, acc_sh, sem):
        cid = lax.axis_index("c")
        sid = lax.axis_index("s")
        base = cid * NH
        pltpu.sync_copy(zeros_hbm, acc_sh.at[pl.ds(sid * RPT, RPT)])
        plsc.subcore_barrier()

        def step(j, carry):
            e0 = sid * ET + j * CH
            pltpu.sync_copy(dst_hbm.at[sid, j], di_v)
            pltpu.sync_copy(vals_hbm.at[pl.ds(e0, CH)], rows_v)
            for g in range(G):
                for i in range(K // 16):
                    v = di_v[g, pl.ds(i * 16, 16)]
                    rel = v - base
                    ok = (rel >= 0) & (rel < NH)
                    ti_v[g, pl.ds(i * 16, 16)] = jnp.where(
                        ok, rel, TRASH + (v & 63))
            for g in range(G):
                pltpu.sync_copy(rows_v.at[pl.ds(g * K, K)],
                                acc_sh.at[ti_v.at[g]], add=True)
            return carry

        lax.fori_loop(0, STEPS_S, step, 0)
        plsc.subcore_barrier()
        pltpu.sync_copy(acc_sh.at[pl.ds(sid * RPT, RPT)],
                        out_hbm.at[pl.ds(cid * NACC + sid * RPT, RPT)])

    return segment_sum2


def _segment_sum2(vals, dst2, zeros_tile):
    return _build_segment_sum2()(vals, dst2, zeros_tile)


def kernel(x, edge_attr, edge_index, ne_w1, ne_b1, ne_w2, ne_b2, ee_w1, ee_b1,
           ee_w2, ee_b2, gn_em_w1, gn_em_b1, gn_em_w2, gn_em_b2, gn_nm_w1,
           gn_nm_b1, gn_nm_w2, gn_nm_b2, de_w1, de_b1, de_w2, de_b2,
           node_mean, node_std, edge_mean, edge_std, out_mean, out_std):
    r1 = lambda v: v.reshape(1, -1)
    src2 = edge_index[0].reshape(NW, STEPS, G, K)
    dst2 = edge_index[1].reshape(NW, STEPS, G, K)
    dst_s = edge_index[1].reshape(NS, STEPS_S, G, K)
    zeros_tile = jnp.zeros((RPT, H), _f32)

    we = [gn_em_w1[i][:H] for i in range(2)]
    ws = [gn_em_w1[i][H:2 * H] for i in range(2)]
    wd = [gn_em_w1[i][2 * H:] for i in range(2)]
    na1 = [gn_nm_w1[i][:H] for i in range(2)]
    na2 = [gn_nm_w1[i][H:] for i in range(2)]

    x0, ps, pd = _node_encode(x, r1(node_mean), r1(node_std), ne_w1, r1(ne_b1),
                              ne_w2, r1(ne_b2), ws[0], r1(gn_em_b1[0]), wd[0])
    e = _edge_encode(edge_attr, r1(edge_mean), r1(edge_std), ee_w1, r1(ee_b1),
                     ee_w2, r1(ee_b2))

    gs, gd = _gather_pair(ps, pd, src2, dst2)
    e = _edge_update(e, gs, gd, we[0], gn_em_w2[0], r1(gn_em_b2[0]))
    parts = _segment_sum2(e, dst_s, zeros_tile)
    agg = jnp.concatenate([parts[:NH], parts[NACC:NACC + NH]], axis=0)
    x1, ps1, pd1 = _node_update(x0, agg, na1[0], na2[0],
                                r1(gn_nm_b1[0]), gn_nm_w2[0], r1(gn_nm_b2[0]),
                                ws[1], r1(gn_em_b1[1]), wd[1])

    gs1, gd1 = _gather_pair(ps1, pd1, src2, dst2)
    e = _edge_update(e, gs1, gd1, we[1], gn_em_w2[1], r1(gn_em_b2[1]))
    parts1 = _segment_sum2(e, dst_s, zeros_tile)
    agg1 = jnp.concatenate([parts1[:NH], parts1[NACC:NACC + NH]], axis=0)

    out_dim = de_w2.shape[1]
    dw2p = jnp.pad(de_w2, ((0, 0), (0, H - out_dim)))
    db2p = jnp.pad(de_b2, (0, H - out_dim))
    stdp = jnp.pad(out_std, (0, H - out_dim), constant_values=1.0)
    meanp = jnp.pad(out_mean, (0, H - out_dim))
    out_full = _node_final(x1, agg1, x0, na1[1], na2[1],
                           r1(gn_nm_b1[1]), gn_nm_w2[1], r1(gn_nm_b2[1]),
                           de_w1, r1(de_b1), dw2p, r1(db2p), r1(stdp),
                           r1(meanp))
    return out_full[:, :out_dim]
